# baseline (speedup 1.0000x reference)
import functools

import jax
import jax.numpy as jnp
from jax.experimental import pallas as pl
from jax.experimental.pallas import tpu as pltpu


def _round_up(n, m):
    return ((n + m - 1) // m) * m


def _pick_tile(npad, cap):
    for t in (cap, 4096, 2048, 1024, 512, 256, 128):
        if t <= cap and t <= npad and npad % t == 0:
            return t
    return 128


def _knn_kernel(row_ref, col_ref, cb_ref, idx_ref, key_ref, *,
                k, n_valid, npad, ibits, ngroups):
    r = row_ref[0]
    c = col_ref[0]
    dn = (((1,), (0,)), ((), ()))
    r_hi = r.astype(jnp.bfloat16)
    r_lo = (r - r_hi.astype(jnp.float32)).astype(jnp.bfloat16)
    c_hi = c.astype(jnp.bfloat16)
    c_lo = (c - c_hi.astype(jnp.float32)).astype(jnp.bfloat16)
    d = (jax.lax.dot_general(r_hi, c_hi, dn, preferred_element_type=jnp.float32)
         + (jax.lax.dot_general(r_hi, c_lo, dn, preferred_element_type=jnp.float32)
            + jax.lax.dot_general(r_lo, c_hi, dn, preferred_element_type=jnp.float32)))
    d = jnp.maximum(d, 0.0)
    bits = pltpu.bitcast(d, jnp.uint32)
    mask = jnp.uint32(((1 << 32) - 1) ^ ((1 << ibits) - 1))
    packed = (bits & mask) + cb_ref[...]
    if npad != n_valid:
        col_ids = jax.lax.broadcasted_iota(jnp.uint32, d.shape, 1)
        packed = jnp.where(col_ids >= jnp.uint32(n_valid),
                           jnp.uint32(0x7F7FFFFF), packed)
    key_ref[...] = pltpu.bitcast(packed, jnp.float32)

    tm = d.shape[0]
    rg = tm // ngroups
    nvr = npad // 128
    lane = jax.lax.broadcasted_iota(jnp.int32, (rg, 128), 1)
    for g in range(ngroups):
        rows = slice(g * rg, (g + 1) * rg)
        sel = jnp.zeros((rg, 128), jnp.float32)
        m = None
        for rd in range(k):
            kv = key_ref[rows, :]
            if rd > 0:
                kv = jnp.where(kv > m, kv, jnp.inf)
            qm = kv[:, 0:128]
            for j in range(1, nvr):
                qm = jnp.minimum(qm, kv[:, j * 128:(j + 1) * 128])
            m = jnp.min(qm, axis=1, keepdims=True)
            sel = jnp.where(lane == rd, m, sel)
        sel_i = pltpu.bitcast(sel, jnp.int32) & jnp.int32((1 << ibits) - 1)
        idx_ref[0, rows, :] = sel_i[:, :k]


def _knn_idx(xyz, k, *, tm=256, ngroups=2):
    B, N, _ = xyz.shape
    npad = _round_up(N, 128)
    tm = _pick_tile(npad, tm)
    ibits = max(12, (npad - 1).bit_length())

    sq = jnp.sum(xyz * xyz, axis=-1, keepdims=True)
    one = jnp.ones_like(sq)
    zero3 = jnp.zeros_like(xyz)
    row_aug = jnp.concatenate([-2.0 * xyz, sq, one, zero3], axis=-1)
    col_aug = jnp.concatenate([xyz, one, sq, zero3], axis=-1)
    if npad != N:
        row_aug = jnp.pad(row_aug, ((0, 0), (0, npad - N), (0, 0)))
        col_aug = jnp.pad(col_aug, ((0, 0), (0, npad - N), (0, 0)))
    col_aug_t = jnp.transpose(col_aug, (0, 2, 1))
    colbias = (jnp.arange(npad, dtype=jnp.uint32) + jnp.uint32(0x00800000))
    colbias = jnp.broadcast_to(colbias[None, :], (tm, npad))

    fn = functools.partial(_knn_kernel, k=k, n_valid=N, npad=npad,
                           ibits=ibits, ngroups=ngroups)
    idx = pl.pallas_call(
        fn,
        out_shape=jax.ShapeDtypeStruct((B, npad, k), jnp.int32),
        grid=(B, npad // tm),
        in_specs=[
            pl.BlockSpec((1, tm, 8), lambda b, i: (b, i, 0)),
            pl.BlockSpec((1, 8, npad), lambda b, i: (b, 0, 0)),
            pl.BlockSpec((tm, npad), lambda b, i: (0, 0)),
        ],
        out_specs=pl.BlockSpec((1, tm, k), lambda b, i: (b, i, 0)),
        scratch_shapes=[pltpu.VMEM((tm, npad), jnp.float32)],
        compiler_params=pltpu.CompilerParams(
            dimension_semantics=("parallel", "parallel"),
            vmem_limit_bytes=48 * 1024 * 1024,
        ),
    )(row_aug, col_aug_t, colbias)
    return idx[:, :N, :]


def _shape_kernel(rel_ref, idxv_ref, wp_ref, ws_ref, shift_ref,
                  out_ref, oidx_ref, *, n_planes):
    xj = rel_ref[0]
    x0, x1, x2 = xj[0], xj[1], xj[2]
    s = x0 * x0 + x1 * x1 + x2 * x2
    inv = jax.lax.rsqrt(jnp.maximum(s, 1e-24))
    kc = s.shape[0]

    sbits = pltpu.bitcast(s, jnp.int32)
    slot = jax.lax.broadcasted_iota(jnp.int32, s.shape, 0)
    u = (sbits & jnp.int32(-16)) | slot
    rank = jnp.zeros(s.shape, jnp.int32)
    for j in range(kc):
        rank = rank + jnp.where(u > u[j:j + 1, :], 1, 0)
    dropped = rank == kc - 1

    bests = []
    for p in range(n_planes):
        pr = wp_ref[p, 0] * x0 + wp_ref[p, 1] * x1 + wp_ref[p, 2] * x2
        val = pr * jnp.abs(pr) * inv
        val = jnp.where(dropped, -jnp.inf, val)
        bests.append(jnp.max(val, axis=0, keepdims=True))
    best = jnp.concatenate(bests, axis=0)
    acc = jax.lax.dot_general(
        ws_ref[...], best, (((1,), (0,)), ((), ())),
        preferred_element_type=jnp.float32,
        precision=jax.lax.Precision.HIGHEST,
    )
    out_ref[0] = jnp.maximum(acc + shift_ref[...], 0.0)

    idxv = idxv_ref[0]
    rows = []
    for r in range(kc):
        rows.append(jnp.sum(jnp.where(rank == r, idxv, 0), axis=0,
                            keepdims=True))
    oidx_ref[0] = jnp.concatenate(rows, axis=0)


def _shape_features(knn_rel, idx_t, wp, ws_folded, shift, *, tile_cap=2048):
    B, C, Kc, N = knn_rel.shape
    P = wp.shape[0]
    S = ws_folded.shape[0]
    npad = _round_up(N, 128)
    if npad != N:
        knn_rel = jnp.pad(knn_rel, ((0, 0), (0, 0), (0, 0), (0, npad - N)))
        idx_t = jnp.pad(idx_t, ((0, 0), (0, 0), (0, npad - N)))
    tn = _pick_tile(npad, tile_cap)
    fn = functools.partial(_shape_kernel, n_planes=P)
    out, oidx = pl.pallas_call(
        fn,
        out_shape=(jax.ShapeDtypeStruct((B, S, npad), jnp.float32),
                   jax.ShapeDtypeStruct((B, Kc, npad), jnp.int32)),
        grid=(B, npad // tn),
        in_specs=[
            pl.BlockSpec((1, C, Kc, tn), lambda b, t: (b, 0, 0, t)),
            pl.BlockSpec((1, Kc, tn), lambda b, t: (b, 0, t)),
            pl.BlockSpec(memory_space=pltpu.MemorySpace.SMEM),
            pl.BlockSpec((S, P), lambda b, t: (0, 0)),
            pl.BlockSpec((S, 1), lambda b, t: (0, 0)),
        ],
        out_specs=(pl.BlockSpec((1, S, tn), lambda b, t: (b, 0, t)),
                   pl.BlockSpec((1, Kc, tn), lambda b, t: (b, 0, t))),
        compiler_params=pltpu.CompilerParams(
            dimension_semantics=("parallel", "parallel"),
            vmem_limit_bytes=32 * 1024 * 1024,
        ),
    )(knn_rel, idx_t, wp, ws_folded, shift)
    if npad != N:
        out, oidx = out[:, :, :N], oidx[:, :, :N]
    return out, oidx


def _forward_chunk(xyz, wp, ws_folded, shift):
    B, N, _ = xyz.shape
    k = 16

    cand = _knn_idx(xyz, k + 1)
    idx_nb = cand[:, :, 1:]
    Kc = k

    xyz_t = jnp.transpose(xyz, (0, 2, 1))
    idx_t = jnp.transpose(idx_nb, (0, 2, 1))
    src = jnp.broadcast_to(xyz_t[:, :, None, :], (B, 3, Kc, N))
    ind = jnp.broadcast_to(idx_t[:, None, :, :], (B, 3, Kc, N))
    knn_rel = jnp.take_along_axis(src, ind, axis=3) - xyz_t[:, :, None, :]

    shapes, oidx = _shape_features(knn_rel, idx_t, wp, ws_folded, shift)
    idx_sorted = jnp.transpose(oidx[:, : k - 1, :], (0, 2, 1))
    idx_out = jnp.concatenate([cand[:, :, :1], idx_sorted], axis=2)
    return shapes, idx_out


def kernel(xyz, wp, ws_folded, shift):
    xyz = xyz.astype(jnp.float32)
    B, N, _ = xyz.shape
    nchunks = 4
    while B % nchunks:
        nchunks -= 1
    cb = B // nchunks
    parts = [_forward_chunk(xyz[i * cb:(i + 1) * cb], wp, ws_folded, shift)
             for i in range(nchunks)]
    shapes = jnp.concatenate([p[0] for p in parts], axis=0)
    idx_out = jnp.concatenate([p[1] for p in parts], axis=0)
    return shapes, xyz, idx_out


# speedup vs baseline: 1.0174x; 1.0174x over previous
import functools

import jax
import jax.numpy as jnp
from jax.experimental import pallas as pl
from jax.experimental.pallas import tpu as pltpu


def _round_up(n, m):
    return ((n + m - 1) // m) * m


def _pick_tile(npad, cap):
    for t in (cap, 4096, 2048, 1024, 512, 256, 128):
        if t <= cap and t <= npad and npad % t == 0:
            return t
    return 128


def _knn_kernel(row_ref, col_ref, cb_ref, idx_ref, key_ref, *,
                k, n_valid, npad, ibits, ngroups, nblk):
    if True:
        tm = key_ref.shape[0]
        rg = tm // ngroups
        nvr = npad // 128
        lane = jax.lax.broadcasted_iota(jnp.int32, (rg, 128), 1)
        for g in range(ngroups):
            rows = slice(g * rg, (g + 1) * rg)
            sel = jnp.zeros((rg, 128), jnp.float32)
            m = None
            for rd in range(k):
                kv = key_ref[rows, :]
                if rd > 0:
                    kv = jnp.where(kv > m, kv, jnp.inf)
                qm = kv[:, 0:128]
                for j in range(1, nvr):
                    qm = jnp.minimum(qm, kv[:, j * 128:(j + 1) * 128])
                m = jnp.min(qm, axis=1, keepdims=True)
                sel = jnp.where(lane == rd, m, sel)
            sel_i = pltpu.bitcast(sel, jnp.int32) & jnp.int32((1 << ibits) - 1)
            idx_ref[0, rows, :] = sel_i[:, :k]

    if True:
        r = row_ref[0]
        c = col_ref[0]
        dn = (((1,), (0,)), ((), ()))
        r_hi = r.astype(jnp.bfloat16)
        r_lo = (r - r_hi.astype(jnp.float32)).astype(jnp.bfloat16)
        c_hi = c.astype(jnp.bfloat16)
        c_lo = (c - c_hi.astype(jnp.float32)).astype(jnp.bfloat16)
        d = (jax.lax.dot_general(r_hi, c_hi, dn, preferred_element_type=jnp.float32)
             + (jax.lax.dot_general(r_hi, c_lo, dn, preferred_element_type=jnp.float32)
                + jax.lax.dot_general(r_lo, c_hi, dn, preferred_element_type=jnp.float32)))
        d = jnp.maximum(d, 0.0)
        bits = pltpu.bitcast(d, jnp.uint32)
        mask = jnp.uint32(((1 << 32) - 1) ^ ((1 << ibits) - 1))
        packed = (bits & mask) + cb_ref[...]
        if npad != n_valid:
            col_ids = jax.lax.broadcasted_iota(jnp.uint32, d.shape, 1)
            packed = jnp.where(col_ids >= jnp.uint32(n_valid),
                               jnp.uint32(0x7F7FFFFF), packed)
        key_ref[...] = pltpu.bitcast(packed, jnp.float32)


def _knn_idx(xyz, k, *, tm=256, ngroups=2):
    B, N, _ = xyz.shape
    npad = _round_up(N, 128)
    tm = _pick_tile(npad, tm)
    ibits = max(12, (npad - 1).bit_length())

    sq = jnp.sum(xyz * xyz, axis=-1, keepdims=True)
    one = jnp.ones_like(sq)
    zero3 = jnp.zeros_like(xyz)
    row_aug = jnp.concatenate([-2.0 * xyz, sq, one, zero3], axis=-1)
    col_aug = jnp.concatenate([xyz, one, sq, zero3], axis=-1)
    if npad != N:
        row_aug = jnp.pad(row_aug, ((0, 0), (0, npad - N), (0, 0)))
        col_aug = jnp.pad(col_aug, ((0, 0), (0, npad - N), (0, 0)))
    col_aug_t = jnp.transpose(col_aug, (0, 2, 1))
    colbias = (jnp.arange(npad, dtype=jnp.uint32) + jnp.uint32(0x00800000))
    colbias = jnp.broadcast_to(colbias[None, :], (tm, npad))

    nblk = npad // tm
    fn = functools.partial(_knn_kernel, k=k, n_valid=N, npad=npad,
                           ibits=ibits, ngroups=ngroups, nblk=nblk)
    idx = pl.pallas_call(
        fn,
        out_shape=jax.ShapeDtypeStruct((B, npad, k), jnp.int32),
        grid=(B, nblk + 1),
        in_specs=[
            pl.BlockSpec((1, tm, 8),
                         lambda b, i: (b, jnp.minimum(i, nblk - 1), 0)),
            pl.BlockSpec((1, 8, npad), lambda b, i: (b, 0, 0)),
            pl.BlockSpec((tm, npad), lambda b, i: (0, 0)),
        ],
        out_specs=pl.BlockSpec((1, tm, k),
                               lambda b, i: (b, jnp.maximum(i - 1, 0), 0)),
        scratch_shapes=[pltpu.VMEM((tm, npad), jnp.float32)],
        compiler_params=pltpu.CompilerParams(
            dimension_semantics=("parallel", "arbitrary"),
            vmem_limit_bytes=48 * 1024 * 1024,
        ),
    )(row_aug, col_aug_t, colbias)
    return idx[:, :N, :]


def _shape_kernel(rel_ref, idxv_ref, wp_ref, ws_ref, shift_ref,
                  out_ref, oidx_ref, *, n_planes):
    xj = rel_ref[0]
    x0, x1, x2 = xj[0], xj[1], xj[2]
    s = x0 * x0 + x1 * x1 + x2 * x2
    inv = jax.lax.rsqrt(jnp.maximum(s, 1e-24))
    kc = s.shape[0]

    sbits = pltpu.bitcast(s, jnp.int32)
    slot = jax.lax.broadcasted_iota(jnp.int32, s.shape, 0)
    u = (sbits & jnp.int32(-16)) | slot
    rank = jnp.zeros(s.shape, jnp.int32)
    for j in range(kc):
        rank = rank + jnp.where(u > u[j:j + 1, :], 1, 0)
    dropped = rank == kc - 1

    bests = []
    for p in range(n_planes):
        pr = wp_ref[p, 0] * x0 + wp_ref[p, 1] * x1 + wp_ref[p, 2] * x2
        val = pr * jnp.abs(pr) * inv
        val = jnp.where(dropped, -jnp.inf, val)
        bests.append(jnp.max(val, axis=0, keepdims=True))
    best = jnp.concatenate(bests, axis=0)
    acc = jax.lax.dot_general(
        ws_ref[...], best, (((1,), (0,)), ((), ())),
        preferred_element_type=jnp.float32,
        precision=jax.lax.Precision.HIGHEST,
    )
    out_ref[0] = jnp.maximum(acc + shift_ref[...], 0.0)

    idxv = idxv_ref[0]
    rows = []
    for r in range(kc):
        rows.append(jnp.sum(jnp.where(rank == r, idxv, 0), axis=0,
                            keepdims=True))
    oidx_ref[0] = jnp.concatenate(rows, axis=0)


def _shape_features(knn_rel, idx_t, wp, ws_folded, shift, *, tile_cap=2048):
    B, C, Kc, N = knn_rel.shape
    P = wp.shape[0]
    S = ws_folded.shape[0]
    npad = _round_up(N, 128)
    if npad != N:
        knn_rel = jnp.pad(knn_rel, ((0, 0), (0, 0), (0, 0), (0, npad - N)))
        idx_t = jnp.pad(idx_t, ((0, 0), (0, 0), (0, npad - N)))
    tn = _pick_tile(npad, tile_cap)
    fn = functools.partial(_shape_kernel, n_planes=P)
    out, oidx = pl.pallas_call(
        fn,
        out_shape=(jax.ShapeDtypeStruct((B, S, npad), jnp.float32),
                   jax.ShapeDtypeStruct((B, Kc, npad), jnp.int32)),
        grid=(B, npad // tn),
        in_specs=[
            pl.BlockSpec((1, C, Kc, tn), lambda b, t: (b, 0, 0, t)),
            pl.BlockSpec((1, Kc, tn), lambda b, t: (b, 0, t)),
            pl.BlockSpec(memory_space=pltpu.MemorySpace.SMEM),
            pl.BlockSpec((S, P), lambda b, t: (0, 0)),
            pl.BlockSpec((S, 1), lambda b, t: (0, 0)),
        ],
        out_specs=(pl.BlockSpec((1, S, tn), lambda b, t: (b, 0, t)),
                   pl.BlockSpec((1, Kc, tn), lambda b, t: (b, 0, t))),
        compiler_params=pltpu.CompilerParams(
            dimension_semantics=("parallel", "parallel"),
            vmem_limit_bytes=32 * 1024 * 1024,
        ),
    )(knn_rel, idx_t, wp, ws_folded, shift)
    if npad != N:
        out, oidx = out[:, :, :N], oidx[:, :, :N]
    return out, oidx


def _forward_chunk(xyz, wp, ws_folded, shift):
    B, N, _ = xyz.shape
    k = 16

    cand = _knn_idx(xyz, k + 1)
    idx_nb = cand[:, :, 1:]
    Kc = k

    xyz_t = jnp.transpose(xyz, (0, 2, 1))
    idx_t = jnp.transpose(idx_nb, (0, 2, 1))
    src = jnp.broadcast_to(xyz_t[:, :, None, :], (B, 3, Kc, N))
    ind = jnp.broadcast_to(idx_t[:, None, :, :], (B, 3, Kc, N))
    knn_rel = jnp.take_along_axis(src, ind, axis=3) - xyz_t[:, :, None, :]

    shapes, oidx = _shape_features(knn_rel, idx_t, wp, ws_folded, shift)
    idx_sorted = jnp.transpose(oidx[:, : k - 1, :], (0, 2, 1))
    idx_out = jnp.concatenate([cand[:, :, :1], idx_sorted], axis=2)
    return shapes, idx_out


def kernel(xyz, wp, ws_folded, shift):
    xyz = xyz.astype(jnp.float32)
    B, N, _ = xyz.shape
    nchunks = 4
    while B % nchunks:
        nchunks -= 1
    cb = B // nchunks
    parts = [_forward_chunk(xyz[i * cb:(i + 1) * cb], wp, ws_folded, shift)
             for i in range(nchunks)]
    shapes = jnp.concatenate([p[0] for p in parts], axis=0)
    idx_out = jnp.concatenate([p[1] for p in parts], axis=0)
    return shapes, xyz, idx_out


# speedup vs baseline: 1.0246x; 1.0071x over previous
import functools

import jax
import jax.numpy as jnp
from jax.experimental import pallas as pl
from jax.experimental.pallas import tpu as pltpu


def _round_up(n, m):
    return ((n + m - 1) // m) * m


def _pick_tile(npad, cap):
    for t in (cap, 4096, 2048, 1024, 512, 256, 128):
        if t <= cap and t <= npad and npad % t == 0:
            return t
    return 128


def _knn_kernel(row_ref, col_ref, cb_ref, idx_ref, key_ref, *,
                k, n_valid, npad, ibits, ngroups, nblk):
    tm = key_ref.shape[0]
    rg = tm // ngroups
    nvr = npad // 128
    lane = jax.lax.broadcasted_iota(jnp.int32, (rg, 128), 1)
    for g in range(ngroups):
        rows = slice(g * rg, (g + 1) * rg)
        sel = jnp.zeros((rg, 128), jnp.float32)
        m = None
        for rd in range(k):
            kv = key_ref[rows, :]
            if rd > 0:
                kv = jnp.where(kv > m, kv, jnp.inf)
            qm = kv[:, 0:128]
            for j in range(1, nvr):
                qm = jnp.minimum(qm, kv[:, j * 128:(j + 1) * 128])
            m = jnp.min(qm, axis=1, keepdims=True)
            sel = jnp.where(lane == rd, m, sel)
        sel_i = pltpu.bitcast(sel, jnp.int32) & jnp.int32((1 << ibits) - 1)
        idx_ref[0, rows, :] = sel_i[:, :k]

    r = row_ref[0]
    c = col_ref[0]
    dn = (((1,), (0,)), ((), ()))
    r_hi = r.astype(jnp.bfloat16)
    r_lo = (r - r_hi.astype(jnp.float32)).astype(jnp.bfloat16)
    c_hi = c.astype(jnp.bfloat16)
    c_lo = (c - c_hi.astype(jnp.float32)).astype(jnp.bfloat16)
    d = (jax.lax.dot_general(r_hi, c_hi, dn, preferred_element_type=jnp.float32)
         + (jax.lax.dot_general(r_hi, c_lo, dn, preferred_element_type=jnp.float32)
            + jax.lax.dot_general(r_lo, c_hi, dn, preferred_element_type=jnp.float32)))
    d = jnp.maximum(d, 0.0)
    bits = pltpu.bitcast(d, jnp.uint32)
    mask = jnp.uint32(((1 << 32) - 1) ^ ((1 << ibits) - 1))
    packed = (bits & mask) + cb_ref[...]
    if npad != n_valid:
        col_ids = jax.lax.broadcasted_iota(jnp.uint32, d.shape, 1)
        packed = jnp.where(col_ids >= jnp.uint32(n_valid),
                           jnp.uint32(0x7F7FFFFF), packed)
    key_ref[...] = pltpu.bitcast(packed, jnp.float32)


def _knn_idx(xyz, k, *, tm=256, ngroups=2):
    B, N, _ = xyz.shape
    npad = _round_up(N, 128)
    tm = _pick_tile(npad, tm)
    ibits = max(12, (npad - 1).bit_length())

    sq = jnp.sum(xyz * xyz, axis=-1, keepdims=True)
    one = jnp.ones_like(sq)
    zero3 = jnp.zeros_like(xyz)
    row_aug = jnp.concatenate([-2.0 * xyz, sq, one, zero3], axis=-1)
    col_aug = jnp.concatenate([xyz, one, sq, zero3], axis=-1)
    if npad != N:
        row_aug = jnp.pad(row_aug, ((0, 0), (0, npad - N), (0, 0)))
        col_aug = jnp.pad(col_aug, ((0, 0), (0, npad - N), (0, 0)))
    col_aug_t = jnp.transpose(col_aug, (0, 2, 1))
    colbias = (jnp.arange(npad, dtype=jnp.uint32) + jnp.uint32(0x00800000))
    colbias = jnp.broadcast_to(colbias[None, :], (tm, npad))

    nblk = npad // tm
    fn = functools.partial(_knn_kernel, k=k, n_valid=N, npad=npad,
                           ibits=ibits, ngroups=ngroups, nblk=nblk)
    idx = pl.pallas_call(
        fn,
        out_shape=jax.ShapeDtypeStruct((B, npad, k), jnp.int32),
        grid=(B, nblk + 1),
        in_specs=[
            pl.BlockSpec((1, tm, 8),
                         lambda b, i: (b, jnp.minimum(i, nblk - 1), 0)),
            pl.BlockSpec((1, 8, npad), lambda b, i: (b, 0, 0)),
            pl.BlockSpec((tm, npad), lambda b, i: (0, 0)),
        ],
        out_specs=pl.BlockSpec((1, tm, k),
                               lambda b, i: (b, jnp.maximum(i - 1, 0), 0)),
        scratch_shapes=[pltpu.VMEM((tm, npad), jnp.float32)],
        compiler_params=pltpu.CompilerParams(
            dimension_semantics=("parallel", "arbitrary"),
            vmem_limit_bytes=48 * 1024 * 1024,
        ),
    )(row_aug, col_aug_t, colbias)
    return idx[:, :N, :]


def _shape_kernel(rel_ref, idxv_ref, wp_ref, ws_ref, shift_ref,
                  out_ref, oidx_ref, *, n_planes):
    xj = rel_ref[0]
    x0, x1, x2 = xj[0], xj[1], xj[2]
    s = x0 * x0 + x1 * x1 + x2 * x2
    inv = jax.lax.rsqrt(jnp.maximum(s, 1e-24))
    kc = s.shape[0]

    sbits = pltpu.bitcast(s, jnp.int32)
    slot = jax.lax.broadcasted_iota(jnp.int32, s.shape, 0)
    u = (sbits & jnp.int32(-16)) | slot
    rank = jnp.zeros(s.shape, jnp.int32)
    for j in range(kc):
        rank = rank + jnp.where(u > u[j:j + 1, :], 1, 0)
    dropped = rank == kc - 1

    bests = []
    for p in range(n_planes):
        pr = wp_ref[p, 0] * x0 + wp_ref[p, 1] * x1 + wp_ref[p, 2] * x2
        val = pr * jnp.abs(pr) * inv
        val = jnp.where(dropped, -jnp.inf, val)
        bests.append(jnp.max(val, axis=0, keepdims=True))
    best = jnp.concatenate(bests, axis=0)
    acc = jax.lax.dot_general(
        ws_ref[...], best, (((1,), (0,)), ((), ())),
        preferred_element_type=jnp.float32,
        precision=jax.lax.Precision.HIGHEST,
    )
    out_ref[0] = jnp.maximum(acc + shift_ref[...], 0.0)

    idxv = idxv_ref[0]
    rows = []
    for r in range(kc):
        rows.append(jnp.sum(jnp.where(rank == r, idxv, 0), axis=0,
                            keepdims=True))
    oidx_ref[0] = jnp.concatenate(rows, axis=0)


def _shape_features(knn_rel, idx_t, wp, ws_folded, shift, *, tile_cap=2048):
    B, C, Kc, N = knn_rel.shape
    P = wp.shape[0]
    S = ws_folded.shape[0]
    npad = _round_up(N, 128)
    if npad != N:
        knn_rel = jnp.pad(knn_rel, ((0, 0), (0, 0), (0, 0), (0, npad - N)))
        idx_t = jnp.pad(idx_t, ((0, 0), (0, 0), (0, npad - N)))
    tn = _pick_tile(npad, tile_cap)
    fn = functools.partial(_shape_kernel, n_planes=P)
    out, oidx = pl.pallas_call(
        fn,
        out_shape=(jax.ShapeDtypeStruct((B, S, npad), jnp.float32),
                   jax.ShapeDtypeStruct((B, Kc, npad), jnp.int32)),
        grid=(B, npad // tn),
        in_specs=[
            pl.BlockSpec((1, C, Kc, tn), lambda b, t: (b, 0, 0, t)),
            pl.BlockSpec((1, Kc, tn), lambda b, t: (b, 0, t)),
            pl.BlockSpec(memory_space=pltpu.MemorySpace.SMEM),
            pl.BlockSpec((S, P), lambda b, t: (0, 0)),
            pl.BlockSpec((S, 1), lambda b, t: (0, 0)),
        ],
        out_specs=(pl.BlockSpec((1, S, tn), lambda b, t: (b, 0, t)),
                   pl.BlockSpec((1, Kc, tn), lambda b, t: (b, 0, t))),
        compiler_params=pltpu.CompilerParams(
            dimension_semantics=("parallel", "parallel"),
            vmem_limit_bytes=32 * 1024 * 1024,
        ),
    )(knn_rel, idx_t, wp, ws_folded, shift)
    if npad != N:
        out, oidx = out[:, :, :N], oidx[:, :, :N]
    return out, oidx


def _forward_chunk(xyz, wp, ws_folded, shift):
    B, N, _ = xyz.shape
    k = 16

    cand = _knn_idx(xyz, k + 1)
    idx_nb = cand[:, :, 1:]
    Kc = k

    xyz_t = jnp.transpose(xyz, (0, 2, 1))
    idx_t = jnp.transpose(idx_nb, (0, 2, 1))
    src = jnp.broadcast_to(xyz_t[:, :, None, :], (B, 3, Kc, N))
    ind = jnp.broadcast_to(idx_t[:, None, :, :], (B, 3, Kc, N))
    knn_rel = jnp.take_along_axis(src, ind, axis=3) - xyz_t[:, :, None, :]

    shapes, oidx = _shape_features(knn_rel, idx_t, wp, ws_folded, shift)
    idx_sorted = jnp.transpose(oidx[:, : k - 1, :], (0, 2, 1))
    idx_out = jnp.concatenate([cand[:, :, :1], idx_sorted], axis=2)
    return shapes, idx_out


def kernel(xyz, wp, ws_folded, shift):
    xyz = xyz.astype(jnp.float32)
    B, N, _ = xyz.shape
    nchunks = 4
    while B % nchunks:
        nchunks -= 1
    cb = B // nchunks
    parts = [_forward_chunk(xyz[i * cb:(i + 1) * cb], wp, ws_folded, shift)
             for i in range(nchunks)]
    shapes = jnp.concatenate([p[0] for p in parts], axis=0)
    idx_out = jnp.concatenate([p[1] for p in parts], axis=0)
    return shapes, xyz, idx_out
